# Initial kernel scaffold; baseline (speedup 1.0000x reference)
import jax
import jax.numpy as jnp
from jax.experimental import pallas as pl


def kernel(x, edge_index, edge_attr, u, batch, W1, b1, W2, b2):
    raise NotImplementedError("write your pallas kernel here")


# speedup vs baseline: 2.3287x; 2.3287x over previous
import functools

import jax
import jax.numpy as jnp
from jax import lax
from jax.experimental import pallas as pl
from jax.experimental.pallas import tpu as pltpu
from jax.experimental.pallas import tpu_sc as plsc

N = 10000
E = 160000
E_PAD = 163840
NODE_DIM = 256
EDGE_DIM = 16
GLOBAL_DIM = 64
HIDDEN_DIM = 512
B_GRAPHS = 64

N_TILES = 32
EDGES_PER_TILE = E_PAD // N_TILES
CHUNK = 128
CHUNKS_PER_TILE = EDGES_PER_TILE // CHUNK
ROWS_PER_TILE = 632
SUM_ROWS = 10112
CNT_ROWS = 640

BLK = 1000
GRID = N // BLK


def _sc_scatter_body(ea_hbm, col_hbm, z_hbm, sums_hbm, counts_hbm,
                     edge_v, idx_v, cnt_v, sums_sh):
    cid = lax.axis_index("c")
    sid = lax.axis_index("s")
    w = cid * 16 + sid

    pltpu.sync_copy(ea_hbm.at[pl.ds(w * EDGES_PER_TILE, EDGES_PER_TILE)], edge_v)
    pltpu.sync_copy(col_hbm.at[pl.ds(w * CHUNKS_PER_TILE, CHUNKS_PER_TILE)], idx_v)
    pltpu.sync_copy(z_hbm, cnt_v)
    pltpu.sync_copy(z_hbm.at[pl.ds(0, ROWS_PER_TILE)],
                    sums_sh.at[pl.ds(sid * ROWS_PER_TILE, ROWS_PER_TILE)])
    plsc.subcore_barrier()

    ones = jnp.full((16,), 1.0, jnp.float32)

    def chunk_step(j, carry):
        pltpu.sync_copy(edge_v.at[pl.ds(j * CHUNK, CHUNK)],
                        sums_sh.at[idx_v.at[j]], add=True)
        for k in range(CHUNK // 16):
            c = idx_v[j, pl.ds(k * 16, 16)]
            row = lax.shift_right_logical(c, 4)
            lane = lax.bitwise_and(c, 15)
            plsc.addupdate_scatter(cnt_v, [row, lane], ones)
        return carry

    lax.fori_loop(0, CHUNKS_PER_TILE, chunk_step, 0)
    plsc.subcore_barrier()

    pltpu.sync_copy(sums_sh.at[pl.ds(sid * ROWS_PER_TILE, ROWS_PER_TILE)],
                    sums_hbm.at[cid, pl.ds(sid * ROWS_PER_TILE, ROWS_PER_TILE)])
    pltpu.sync_copy(cnt_v, counts_hbm.at[w])


@functools.cache
def _get_sc_scatter():
    return functools.partial(
        pl.kernel,
        out_type=[
            jax.ShapeDtypeStruct((2, SUM_ROWS, EDGE_DIM), jnp.float32),
            jax.ShapeDtypeStruct((N_TILES, CNT_ROWS, 16), jnp.float32),
        ],
        mesh=plsc.VectorSubcoreMesh(core_axis_name="c", subcore_axis_name="s",
                                    num_cores=2, num_subcores=16),
        scratch_types=[
            pltpu.VMEM((EDGES_PER_TILE, EDGE_DIM), jnp.float32),
            pltpu.VMEM((CHUNKS_PER_TILE, CHUNK), jnp.int32),
            pltpu.VMEM((CNT_ROWS, 16), jnp.float32),
            pltpu.VMEM_SHARED((SUM_ROWS, EDGE_DIM), jnp.float32),
        ],
        compiler_params=pltpu.CompilerParams(needs_layout_passes=False,
                                             use_tc_tiling_on_sc=False),
    )(_sc_scatter_body)


def _tc_mlp_body(x_ref, s0_ref, s1_ref, cnt_ref, b_ref, u_ref,
                 w1x_ref, w1a_ref, w1u_ref, b1_ref, w2_ref, b2_ref, o_ref):
    cnt = jnp.sum(cnt_ref[...], axis=0)
    agg = (s0_ref[...] + s1_ref[...]) / jnp.maximum(cnt, 1.0)
    oh = (b_ref[...] == lax.broadcasted_iota(jnp.int32, (BLK, B_GRAPHS), 1)
          ).astype(jnp.float32)
    uw = jnp.dot(u_ref[...], w1u_ref[...], preferred_element_type=jnp.float32)
    acc = jnp.dot(x_ref[...], w1x_ref[...], preferred_element_type=jnp.float32)
    acc = acc + jnp.dot(agg, w1a_ref[...], preferred_element_type=jnp.float32)
    acc = acc + jnp.dot(oh, uw, preferred_element_type=jnp.float32)
    h1 = jnp.maximum(acc + b1_ref[...], 0.0)
    o_ref[...] = (jnp.dot(h1, w2_ref[...], preferred_element_type=jnp.float32)
                  + b2_ref[...])


def _tc_mlp(x, s0, s1, cnt, batch2d, u, w1x, w1a, w1u, b1r, w2, b2r):
    return pl.pallas_call(
        _tc_mlp_body,
        grid=(GRID,),
        in_specs=[
            pl.BlockSpec((BLK, NODE_DIM), lambda i: (i, 0)),
            pl.BlockSpec((BLK, EDGE_DIM), lambda i: (i, 0)),
            pl.BlockSpec((BLK, EDGE_DIM), lambda i: (i, 0)),
            pl.BlockSpec((N_TILES, BLK, 1), lambda i: (0, i, 0)),
            pl.BlockSpec((BLK, 1), lambda i: (i, 0)),
            pl.BlockSpec((B_GRAPHS, GLOBAL_DIM), lambda i: (0, 0)),
            pl.BlockSpec((NODE_DIM, HIDDEN_DIM), lambda i: (0, 0)),
            pl.BlockSpec((EDGE_DIM, HIDDEN_DIM), lambda i: (0, 0)),
            pl.BlockSpec((GLOBAL_DIM, HIDDEN_DIM), lambda i: (0, 0)),
            pl.BlockSpec((1, HIDDEN_DIM), lambda i: (0, 0)),
            pl.BlockSpec((HIDDEN_DIM, NODE_DIM), lambda i: (0, 0)),
            pl.BlockSpec((1, NODE_DIM), lambda i: (0, 0)),
        ],
        out_specs=pl.BlockSpec((BLK, NODE_DIM), lambda i: (i, 0)),
        out_shape=jax.ShapeDtypeStruct((N, NODE_DIM), jnp.float32),
        compiler_params=pltpu.CompilerParams(
            dimension_semantics=("arbitrary",)),
    )(x, s0, s1, cnt, batch2d, u, w1x, w1a, w1u, b1r, w2, b2r)


def kernel(x, edge_index, edge_attr, u, batch, W1, b1, W2, b2):
    col = edge_index[1].astype(jnp.int32)
    n_pad = E_PAD - E
    pad_col = (N + (jnp.arange(n_pad, dtype=jnp.int32) % 16))
    col_pad = jnp.concatenate([col, pad_col]).reshape(E_PAD // CHUNK, CHUNK)
    ea_pad = jnp.concatenate(
        [edge_attr, jnp.zeros((n_pad, EDGE_DIM), jnp.float32)])
    zeros = jnp.zeros((CNT_ROWS, 16), jnp.float32)

    sums, counts = _get_sc_scatter()(ea_pad, col_pad, zeros)

    s0 = sums[0, :N, :]
    s1 = sums[1, :N, :]
    cnt = counts.reshape(N_TILES, CNT_ROWS * 16)[:, :N].reshape(N_TILES, N, 1)
    batch2d = batch.astype(jnp.int32).reshape(N, 1)
    w1x = W1[:NODE_DIM]
    w1a = W1[NODE_DIM:NODE_DIM + EDGE_DIM]
    w1u = W1[NODE_DIM + EDGE_DIM:]
    b1r = b1.reshape(1, HIDDEN_DIM)
    b2r = b2.reshape(1, NODE_DIM)
    return _tc_mlp(x, s0, s1, cnt, batch2d, u, w1x, w1a, w1u, b1r, W2, b2r)


# speedup vs baseline: 4.7688x; 2.0479x over previous
import functools

import jax
import jax.numpy as jnp
from jax import lax
from jax.experimental import pallas as pl
from jax.experimental.pallas import tpu as pltpu
from jax.experimental.pallas import tpu_sc as plsc

N = 10000
E = 160000
NODE_DIM = 256
EDGE_DIM = 16
GLOBAL_DIM = 64
HIDDEN_DIM = 512
B_GRAPHS = 64

N_TILES = 32
CHUNK = 128
N_CHUNKS = E // CHUNK
BASE_ROWS = N_CHUNKS // N_TILES
MAX_ROWS = BASE_ROWS + 1
ROWS_PER_TILE = 640
SUM_ROWS = 10240

BLK = 1000
GRID = N // BLK
AGG_BLK = 640
AGG_GRID = SUM_ROWS // AGG_BLK


def _sc_scatter_body(ea_hbm, col_hbm, z_hbm, sums_hbm, cnt_hbm,
                     edge_v, idx_v, cnt_v, sums_sh):
    cid = lax.axis_index("c")
    sid = lax.axis_index("s")
    w = cid * 16 + sid
    base = w * BASE_ROWS + jnp.minimum(w, 2)
    nrows = jnp.where(w < 2, MAX_ROWS, BASE_ROWS)
    dma_base = jnp.minimum(base, N_CHUNKS - MAX_ROWS)
    off = base - dma_base

    pltpu.sync_copy(ea_hbm.at[pl.ds(dma_base * CHUNK, MAX_ROWS * CHUNK)],
                    edge_v)
    pltpu.sync_copy(col_hbm.at[pl.ds(dma_base, MAX_ROWS)], idx_v)
    pltpu.sync_copy(z_hbm, sums_sh.at[pl.ds(sid * ROWS_PER_TILE,
                                            ROWS_PER_TILE)])
    zeros16 = jnp.zeros((16,), jnp.float32)

    def zero_step(i, carry):
        cnt_v[pl.ds(i * 16, 16)] = zeros16
        return carry

    lax.fori_loop(0, SUM_ROWS // 16, zero_step, 0)
    plsc.subcore_barrier()

    ones = jnp.full((16,), 1.0, jnp.float32)

    def chunk_step(j, carry):
        row = off + j
        pltpu.sync_copy(edge_v.at[pl.ds(row * CHUNK, CHUNK)],
                        sums_sh.at[idx_v.at[row]], add=True)
        for k in range(CHUNK // 16):
            c = idx_v[row, pl.ds(k * 16, 16)]
            plsc.addupdate_scatter(cnt_v, [c], ones)
        return carry

    lax.fori_loop(0, nrows, chunk_step, 0)
    plsc.subcore_barrier()

    pltpu.sync_copy(sums_sh.at[pl.ds(sid * ROWS_PER_TILE, ROWS_PER_TILE)],
                    sums_hbm.at[cid, pl.ds(sid * ROWS_PER_TILE,
                                           ROWS_PER_TILE)])
    pltpu.sync_copy(cnt_v, cnt_hbm.at[w])


@functools.cache
def _get_sc_scatter():
    return functools.partial(
        pl.kernel,
        out_type=[
            jax.ShapeDtypeStruct((2, SUM_ROWS, EDGE_DIM), jnp.float32),
            jax.ShapeDtypeStruct((N_TILES, SUM_ROWS), jnp.float32),
        ],
        mesh=plsc.VectorSubcoreMesh(core_axis_name="c", subcore_axis_name="s",
                                    num_cores=2, num_subcores=16),
        scratch_types=[
            pltpu.VMEM((MAX_ROWS * CHUNK, EDGE_DIM), jnp.float32),
            pltpu.VMEM((MAX_ROWS, CHUNK), jnp.int32),
            pltpu.VMEM((SUM_ROWS,), jnp.float32),
            pltpu.VMEM_SHARED((SUM_ROWS, EDGE_DIM), jnp.float32),
        ],
        compiler_params=pltpu.CompilerParams(needs_layout_passes=False,
                                             use_tc_tiling_on_sc=False),
    )(_sc_scatter_body)


def _agg_body(s0_ref, s1_ref, cnt_ref, o_ref):
    ones = jnp.ones((N_TILES, 1), jnp.float32)
    c = lax.dot_general(cnt_ref[...], ones, (((0,), (0,)), ((), ())),
                        preferred_element_type=jnp.float32)
    o_ref[...] = (s0_ref[...] + s1_ref[...]) / jnp.maximum(c, 1.0)


def _agg(s0, s1, cnt):
    return pl.pallas_call(
        _agg_body,
        grid=(AGG_GRID,),
        in_specs=[
            pl.BlockSpec((AGG_BLK, EDGE_DIM), lambda i: (i, 0)),
            pl.BlockSpec((AGG_BLK, EDGE_DIM), lambda i: (i, 0)),
            pl.BlockSpec((N_TILES, AGG_BLK), lambda i: (0, i)),
        ],
        out_specs=pl.BlockSpec((AGG_BLK, EDGE_DIM), lambda i: (i, 0)),
        out_shape=jax.ShapeDtypeStruct((SUM_ROWS, EDGE_DIM), jnp.float32),
        compiler_params=pltpu.CompilerParams(
            dimension_semantics=("arbitrary",)),
    )(s0, s1, cnt)


def _tc_mlp_body(x_ref, agg_ref, b_ref, u_ref,
                 w1x_ref, w1a_ref, w1u_ref, b1_ref, w2_ref, b2_ref, o_ref):
    oh = (b_ref[...] == lax.broadcasted_iota(jnp.int32, (BLK, B_GRAPHS), 1)
          ).astype(jnp.float32)
    uw = jnp.dot(u_ref[...], w1u_ref[...], preferred_element_type=jnp.float32)
    acc = jnp.dot(x_ref[...], w1x_ref[...], preferred_element_type=jnp.float32)
    acc = acc + jnp.dot(agg_ref[...], w1a_ref[...],
                        preferred_element_type=jnp.float32)
    acc = acc + jnp.dot(oh, uw, preferred_element_type=jnp.float32)
    h1 = jnp.maximum(acc + b1_ref[...], 0.0)
    o_ref[...] = (jnp.dot(h1, w2_ref[...], preferred_element_type=jnp.float32)
                  + b2_ref[...])


def _tc_mlp(x, agg, batch2d, u, w1x, w1a, w1u, b1r, w2, b2r):
    return pl.pallas_call(
        _tc_mlp_body,
        grid=(GRID,),
        in_specs=[
            pl.BlockSpec((BLK, NODE_DIM), lambda i: (i, 0)),
            pl.BlockSpec((BLK, EDGE_DIM), lambda i: (i, 0)),
            pl.BlockSpec((BLK, 1), lambda i: (i, 0)),
            pl.BlockSpec((B_GRAPHS, GLOBAL_DIM), lambda i: (0, 0)),
            pl.BlockSpec((NODE_DIM, HIDDEN_DIM), lambda i: (0, 0)),
            pl.BlockSpec((EDGE_DIM, HIDDEN_DIM), lambda i: (0, 0)),
            pl.BlockSpec((GLOBAL_DIM, HIDDEN_DIM), lambda i: (0, 0)),
            pl.BlockSpec((1, HIDDEN_DIM), lambda i: (0, 0)),
            pl.BlockSpec((HIDDEN_DIM, NODE_DIM), lambda i: (0, 0)),
            pl.BlockSpec((1, NODE_DIM), lambda i: (0, 0)),
        ],
        out_specs=pl.BlockSpec((BLK, NODE_DIM), lambda i: (i, 0)),
        out_shape=jax.ShapeDtypeStruct((N, NODE_DIM), jnp.float32),
        compiler_params=pltpu.CompilerParams(
            dimension_semantics=("arbitrary",)),
    )(x, agg, batch2d, u, w1x, w1a, w1u, b1r, w2, b2r)


def kernel(x, edge_index, edge_attr, u, batch, W1, b1, W2, b2):
    col = edge_index[1].astype(jnp.int32)
    col2d = col.reshape(N_CHUNKS, CHUNK)
    z = jnp.zeros((ROWS_PER_TILE, EDGE_DIM), jnp.float32)

    sums, cnt = _get_sc_scatter()(edge_attr, col2d, z)

    agg = _agg(sums[0], sums[1], cnt)[:N]
    batch2d = batch.astype(jnp.int32).reshape(N, 1)
    w1x = W1[:NODE_DIM]
    w1a = W1[NODE_DIM:NODE_DIM + EDGE_DIM]
    w1u = W1[NODE_DIM + EDGE_DIM:]
    b1r = b1.reshape(1, HIDDEN_DIM)
    b2r = b2.reshape(1, NODE_DIM)
    return _tc_mlp(x, agg, batch2d, u, w1x, w1a, w1u, b1r, W2, b2r)


# speedup vs baseline: 5.3236x; 1.1163x over previous
import functools

import jax
import jax.numpy as jnp
from jax import lax
from jax.experimental import pallas as pl
from jax.experimental.pallas import tpu as pltpu
from jax.experimental.pallas import tpu_sc as plsc

N = 10000
E = 160000
NODE_DIM = 256
EDGE_DIM = 16
GLOBAL_DIM = 64
HIDDEN_DIM = 512
B_GRAPHS = 64

N_TILES = 32
CHUNK = 128
N_CHUNKS = E // CHUNK
BASE_ROWS = N_CHUNKS // N_TILES
MAX_ROWS = BASE_ROWS + 1
ROWS_PER_TILE = 640
SUM_ROWS = 10240

BLK = 1024
GRID = (N + BLK - 1) // BLK


def _sc_scatter_body(ea_hbm, col_hbm, z_hbm, sums_hbm, cnt_hbm,
                     edge_v, idx_v, cnt_v, sums_sh):
    cid = lax.axis_index("c")
    sid = lax.axis_index("s")
    w = cid * 16 + sid
    base = w * BASE_ROWS + jnp.minimum(w, 2)
    nrows = jnp.where(w < 2, MAX_ROWS, BASE_ROWS)
    dma_base = jnp.minimum(base, N_CHUNKS - MAX_ROWS)
    off = base - dma_base

    pltpu.sync_copy(ea_hbm.at[pl.ds(dma_base * CHUNK, MAX_ROWS * CHUNK)],
                    edge_v)
    pltpu.sync_copy(col_hbm.at[pl.ds(dma_base, MAX_ROWS)], idx_v)
    pltpu.sync_copy(z_hbm, sums_sh.at[pl.ds(sid * ROWS_PER_TILE,
                                            ROWS_PER_TILE)])
    zeros16 = jnp.zeros((16,), jnp.float32)

    def zero_step(i, carry):
        cnt_v[pl.ds(i * 16, 16)] = zeros16
        return carry

    lax.fori_loop(0, SUM_ROWS // 16, zero_step, 0)
    plsc.subcore_barrier()

    ones = jnp.full((16,), 1.0, jnp.float32)

    def chunk_step(j, carry):
        row = off + j
        pltpu.sync_copy(edge_v.at[pl.ds(row * CHUNK, CHUNK)],
                        sums_sh.at[idx_v.at[row]], add=True)
        for k in range(CHUNK // 16):
            c = idx_v[row, pl.ds(k * 16, 16)]
            plsc.addupdate_scatter(cnt_v, [c], ones)
        return carry

    lax.fori_loop(0, nrows, chunk_step, 0)
    plsc.subcore_barrier()

    pltpu.sync_copy(sums_sh.at[pl.ds(sid * ROWS_PER_TILE, ROWS_PER_TILE)],
                    sums_hbm.at[cid, pl.ds(sid * ROWS_PER_TILE,
                                           ROWS_PER_TILE)])
    pltpu.sync_copy(cnt_v, cnt_hbm.at[w])


@functools.cache
def _get_sc_scatter():
    return functools.partial(
        pl.kernel,
        out_type=[
            jax.ShapeDtypeStruct((2, SUM_ROWS, EDGE_DIM), jnp.float32),
            jax.ShapeDtypeStruct((N_TILES, SUM_ROWS), jnp.float32),
        ],
        mesh=plsc.VectorSubcoreMesh(core_axis_name="c", subcore_axis_name="s",
                                    num_cores=2, num_subcores=16),
        scratch_types=[
            pltpu.VMEM((MAX_ROWS * CHUNK, EDGE_DIM), jnp.float32),
            pltpu.VMEM((MAX_ROWS, CHUNK), jnp.int32),
            pltpu.VMEM((SUM_ROWS,), jnp.float32),
            pltpu.VMEM_SHARED((SUM_ROWS, EDGE_DIM), jnp.float32),
        ],
        compiler_params=pltpu.CompilerParams(needs_layout_passes=False,
                                             use_tc_tiling_on_sc=False),
    )(_sc_scatter_body)


def _tc_mlp_body(x_ref, s0_ref, s1_ref, cnt_ref, b_ref, u_ref,
                 w1x_ref, w1a_ref, w1u_ref, b1_ref, w2_ref, b2_ref, o_ref):
    ones = jnp.ones((N_TILES, 1), jnp.float32)
    c = lax.dot_general(cnt_ref[...], ones, (((0,), (0,)), ((), ())),
                        preferred_element_type=jnp.float32)
    inv = 1.0 / jnp.maximum(c, 1.0)
    oh = (b_ref[...] == lax.broadcasted_iota(jnp.int32, (BLK, B_GRAPHS), 1)
          ).astype(jnp.float32)
    uw = jnp.dot(u_ref[...], w1u_ref[...], preferred_element_type=jnp.float32)
    acc = jnp.dot(x_ref[...], w1x_ref[...], preferred_element_type=jnp.float32)
    acc = acc + jnp.dot(s0_ref[...] + s1_ref[...], w1a_ref[...],
                        preferred_element_type=jnp.float32) * inv
    acc = acc + jnp.dot(oh, uw, preferred_element_type=jnp.float32)
    h1 = jnp.maximum(acc + b1_ref[...], 0.0)
    o_ref[...] = (jnp.dot(h1, w2_ref[...], preferred_element_type=jnp.float32)
                  + b2_ref[...])


def _tc_mlp(x, s0, s1, cnt, batch2d, u, w1x, w1a, w1u, b1r, w2, b2r):
    return pl.pallas_call(
        _tc_mlp_body,
        grid=(GRID,),
        in_specs=[
            pl.BlockSpec((BLK, NODE_DIM), lambda i: (i, 0)),
            pl.BlockSpec((BLK, EDGE_DIM), lambda i: (i, 0)),
            pl.BlockSpec((BLK, EDGE_DIM), lambda i: (i, 0)),
            pl.BlockSpec((N_TILES, BLK), lambda i: (0, i)),
            pl.BlockSpec((BLK, 1), lambda i: (i, 0)),
            pl.BlockSpec((B_GRAPHS, GLOBAL_DIM), lambda i: (0, 0)),
            pl.BlockSpec((NODE_DIM, HIDDEN_DIM), lambda i: (0, 0)),
            pl.BlockSpec((EDGE_DIM, HIDDEN_DIM), lambda i: (0, 0)),
            pl.BlockSpec((GLOBAL_DIM, HIDDEN_DIM), lambda i: (0, 0)),
            pl.BlockSpec((1, HIDDEN_DIM), lambda i: (0, 0)),
            pl.BlockSpec((HIDDEN_DIM, NODE_DIM), lambda i: (0, 0)),
            pl.BlockSpec((1, NODE_DIM), lambda i: (0, 0)),
        ],
        out_specs=pl.BlockSpec((BLK, NODE_DIM), lambda i: (i, 0)),
        out_shape=jax.ShapeDtypeStruct((N, NODE_DIM), jnp.float32),
        compiler_params=pltpu.CompilerParams(
            dimension_semantics=("arbitrary",)),
    )(x, s0, s1, cnt, batch2d, u, w1x, w1a, w1u, b1r, w2, b2r)


def kernel(x, edge_index, edge_attr, u, batch, W1, b1, W2, b2):
    col = edge_index[1].astype(jnp.int32)
    col2d = col.reshape(N_CHUNKS, CHUNK)
    z = jnp.zeros((ROWS_PER_TILE, EDGE_DIM), jnp.float32)

    sums, cnt = _get_sc_scatter()(edge_attr, col2d, z)

    s0 = sums[0]
    s1 = sums[1]
    batch2d = batch.astype(jnp.int32).reshape(N, 1)
    w1x = W1[:NODE_DIM]
    w1a = W1[NODE_DIM:NODE_DIM + EDGE_DIM]
    w1u = W1[NODE_DIM + EDGE_DIM:]
    b1r = b1.reshape(1, HIDDEN_DIM)
    b2r = b2.reshape(1, NODE_DIM)
    return _tc_mlp(x, s0, s1, cnt, batch2d, u, w1x, w1a, w1u, b1r, W2, b2r)


# speedup vs baseline: 5.6027x; 1.0524x over previous
import functools

import jax
import jax.numpy as jnp
from jax import lax
from jax.experimental import pallas as pl
from jax.experimental.pallas import tpu as pltpu
from jax.experimental.pallas import tpu_sc as plsc

N = 10000
E = 160000
NODE_DIM = 256
EDGE_DIM = 16
GLOBAL_DIM = 64
HIDDEN_DIM = 512
B_GRAPHS = 64

N_TILES = 32
CHUNK = 128
N_CHUNKS = E // CHUNK
COL_ROWS = 1280
BASE_ROWS = N_CHUNKS // N_TILES
MAX_ROWS = BASE_ROWS + 1
ROWS_PER_TILE = 640
SUM_ROWS = 10240

BLK = 1024
GRID = (N + BLK - 1) // BLK


def _sc_scatter_body(ea_hbm, col_hbm, z_hbm, sums_hbm, cnt_hbm,
                     edge_v, idx_v, cnt_v, sums_sh):
    cid = lax.axis_index("c")
    sid = lax.axis_index("s")
    w = cid * 16 + sid
    base = w * BASE_ROWS + jnp.minimum(w, 2)
    nrows = jnp.where(w < 2, MAX_ROWS, BASE_ROWS)
    dma_base = jnp.minimum(base, N_CHUNKS - MAX_ROWS)
    off = base - dma_base

    pltpu.sync_copy(ea_hbm.at[pl.ds(dma_base * CHUNK, MAX_ROWS * CHUNK)],
                    edge_v)
    pltpu.sync_copy(col_hbm.at[pl.ds(dma_base, MAX_ROWS)], idx_v)
    pltpu.sync_copy(z_hbm, sums_sh.at[pl.ds(sid * ROWS_PER_TILE,
                                            ROWS_PER_TILE)])
    zeros16 = jnp.zeros((16,), jnp.float32)

    def zero_step(i, carry):
        cnt_v[pl.ds(i * 16, 16)] = zeros16
        return carry

    lax.fori_loop(0, SUM_ROWS // 16, zero_step, 0)
    plsc.subcore_barrier()

    ones = jnp.full((16,), 1.0, jnp.float32)

    def chunk_step(j, carry):
        row = off + j
        pltpu.sync_copy(edge_v.at[pl.ds(row * CHUNK, CHUNK)],
                        sums_sh.at[idx_v.at[row]], add=True)
        for k in range(CHUNK // 16):
            c = idx_v[row, pl.ds(k * 16, 16)]
            plsc.addupdate_scatter(cnt_v, [c], ones)
        return carry

    lax.fori_loop(0, nrows, chunk_step, 0)
    plsc.subcore_barrier()

    pltpu.sync_copy(sums_sh.at[pl.ds(sid * ROWS_PER_TILE, ROWS_PER_TILE)],
                    sums_hbm.at[cid, pl.ds(sid * ROWS_PER_TILE,
                                           ROWS_PER_TILE)])
    pltpu.sync_copy(cnt_v, cnt_hbm.at[w])


@functools.cache
def _get_sc_scatter():
    return functools.partial(
        pl.kernel,
        out_type=[
            jax.ShapeDtypeStruct((2, SUM_ROWS, EDGE_DIM), jnp.float32),
            jax.ShapeDtypeStruct((N_TILES, SUM_ROWS), jnp.float32),
        ],
        mesh=plsc.VectorSubcoreMesh(core_axis_name="c", subcore_axis_name="s",
                                    num_cores=2, num_subcores=16),
        scratch_types=[
            pltpu.VMEM((MAX_ROWS * CHUNK, EDGE_DIM), jnp.float32),
            pltpu.VMEM((MAX_ROWS, CHUNK), jnp.int32),
            pltpu.VMEM((SUM_ROWS,), jnp.float32),
            pltpu.VMEM_SHARED((SUM_ROWS, EDGE_DIM), jnp.float32),
        ],
        compiler_params=pltpu.CompilerParams(needs_layout_passes=False,
                                             use_tc_tiling_on_sc=False),
    )(_sc_scatter_body)


def _tc_mlp_body(x_ref, s_ref, cnt_ref, b_ref, u_ref,
                 w1x_ref, w1a_ref, w1u_ref, b1_ref, w2_ref, b2_ref, o_ref):
    ones = jnp.ones((N_TILES, 1), jnp.float32)
    c = lax.dot_general(cnt_ref[...], ones, (((0,), (0,)), ((), ())),
                        preferred_element_type=jnp.float32)
    inv = 1.0 / jnp.maximum(c, 1.0)
    oh = (b_ref[...] == lax.broadcasted_iota(jnp.int32, (BLK, B_GRAPHS), 1)
          ).astype(jnp.float32)
    uw = jnp.dot(u_ref[...], w1u_ref[...], preferred_element_type=jnp.float32)
    acc = jnp.dot(x_ref[...], w1x_ref[...], preferred_element_type=jnp.float32)
    acc = acc + jnp.dot(s_ref[0] + s_ref[1], w1a_ref[...],
                        preferred_element_type=jnp.float32) * inv
    acc = acc + jnp.dot(oh, uw, preferred_element_type=jnp.float32)
    h1 = jnp.maximum(acc + b1_ref[...], 0.0)
    o_ref[...] = (jnp.dot(h1, w2_ref[...], preferred_element_type=jnp.float32)
                  + b2_ref[...])


def _tc_mlp(x, s, cnt, batch2d, u, w1x, w1a, w1u, b1r, w2, b2r):
    return pl.pallas_call(
        _tc_mlp_body,
        grid=(GRID,),
        in_specs=[
            pl.BlockSpec((BLK, NODE_DIM), lambda i: (i, 0)),
            pl.BlockSpec((2, BLK, EDGE_DIM), lambda i: (0, i, 0)),
            pl.BlockSpec((N_TILES, BLK), lambda i: (0, i)),
            pl.BlockSpec((BLK, 1), lambda i: (i, 0)),
            pl.BlockSpec((B_GRAPHS, GLOBAL_DIM), lambda i: (0, 0)),
            pl.BlockSpec((NODE_DIM, HIDDEN_DIM), lambda i: (0, 0)),
            pl.BlockSpec((EDGE_DIM, HIDDEN_DIM), lambda i: (0, 0)),
            pl.BlockSpec((GLOBAL_DIM, HIDDEN_DIM), lambda i: (0, 0)),
            pl.BlockSpec((1, HIDDEN_DIM), lambda i: (0, 0)),
            pl.BlockSpec((HIDDEN_DIM, NODE_DIM), lambda i: (0, 0)),
            pl.BlockSpec((1, NODE_DIM), lambda i: (0, 0)),
        ],
        out_specs=pl.BlockSpec((BLK, NODE_DIM), lambda i: (i, 0)),
        out_shape=jax.ShapeDtypeStruct((N, NODE_DIM), jnp.float32),
        compiler_params=pltpu.CompilerParams(
            dimension_semantics=("arbitrary",)),
    )(x, s, cnt, batch2d, u, w1x, w1a, w1u, b1r, w2, b2r)


def kernel(x, edge_index, edge_attr, u, batch, W1, b1, W2, b2):
    col = edge_index[1].astype(jnp.int32)
    col2d = jnp.concatenate(
        [col, jnp.zeros(((COL_ROWS - N_CHUNKS) * CHUNK,), jnp.int32)]
    ).reshape(COL_ROWS, CHUNK)
    z = jnp.zeros((ROWS_PER_TILE, EDGE_DIM), jnp.float32)

    sums, cnt = _get_sc_scatter()(edge_attr, col2d, z)

    batch2d = batch.astype(jnp.int32).reshape(N, 1)
    w1x = W1[:NODE_DIM]
    w1a = W1[NODE_DIM:NODE_DIM + EDGE_DIM]
    w1u = W1[NODE_DIM + EDGE_DIM:]
    b1r = b1.reshape(1, HIDDEN_DIM)
    b2r = b2.reshape(1, NODE_DIM)
    return _tc_mlp(x, sums, cnt, batch2d, u, w1x, w1a, w1u, b1r, W2, b2r)
